# Initial kernel scaffold; baseline (speedup 1.0000x reference)
import jax
import jax.numpy as jnp
from jax.experimental import pallas as pl


def kernel(x, table):
    raise NotImplementedError("write your pallas kernel here")


# speedup vs baseline: 1.8329x; 1.8329x over previous
import functools

import jax
import jax.numpy as jnp
from jax import lax
from jax.experimental import pallas as pl
from jax.experimental.pallas import tpu as pltpu
from jax.experimental.pallas import tpu_sc as plsc

NC = 2
NS = 16
NW = NC * NS


def _make_emb(V, D, B):
    assert B % NW == 0
    bpw = B // NW
    CH = 8
    NB = 2
    nchunk = bpw // CH
    assert nchunk % NB == 0 and nchunk // NB >= 2

    mesh = plsc.VectorSubcoreMesh(core_axis_name="c", subcore_axis_name="s")

    @functools.partial(
        pl.kernel,
        mesh=mesh,
        out_type=jax.ShapeDtypeStruct((B, D), jnp.float32),
        scratch_types=[
            pltpu.VMEM((bpw,), jnp.int32),
            pltpu.VMEM((NB, CH, D), jnp.float32),
            pltpu.SemaphoreType.DMA,
            pltpu.SemaphoreType.DMA,
            pltpu.SemaphoreType.DMA,
            pltpu.SemaphoreType.DMA,
        ],
    )
    def emb(table_hbm, idx_hbm, out_hbm, idx_v, rows_v, g0, g1, o0, o1):
        wid = lax.axis_index("s") * NC + lax.axis_index("c")
        base = wid * bpw
        gsem = (g0, g1)
        osem = (o0, o1)

        pltpu.sync_copy(idx_hbm.at[pl.ds(base, bpw)], idx_v)

        def gather_desc(c, b):
            return pltpu.make_async_copy(
                table_hbm.at[idx_v.at[pl.ds(c * CH, CH)]],
                rows_v.at[b],
                gsem[b],
            )

        def out_desc(c, b):
            return pltpu.make_async_copy(
                rows_v.at[b],
                out_hbm.at[pl.ds(base + c * CH, CH)],
                osem[b],
            )

        for b in range(NB):
            gather_desc(b, b).start()

        @pl.loop(0, nchunk - NB, step=NB)
        def _(g):
            for b in range(NB):
                c = g + b
                gather_desc(c, b).wait()
                out_desc(c, b).start()
                out_desc(c, b).wait()
                gather_desc(c + NB, b).start()

        for b in range(NB):
            c = nchunk - NB + b
            gather_desc(c, b).wait()
            out_desc(c, b).start()
            out_desc(c, b).wait()

    return emb


def kernel(x, table):
    V, D = table.shape
    B = x.size
    emb = _make_emb(V, D, B)
    out = emb(table, x.reshape(-1))
    return out.reshape(*x.shape, D)
